# Initial kernel scaffold; baseline (speedup 1.0000x reference)
import jax
import jax.numpy as jnp
from jax.experimental import pallas as pl


def kernel(x_pred, x_true, R_pred, R_true):
    raise NotImplementedError("write your pallas kernel here")


# speedup vs baseline: 1.1958x; 1.1958x over previous
import functools

import jax
import jax.numpy as jnp
from jax.experimental import pallas as pl

N_GRID = 128
DT = 5e-05
DX = 1.0 / N_GRID
INV_DX = float(N_GRID)
E_MOD = 1200.0
NU = 0.3
LA = E_MOD * NU / ((1 + NU) * (1 - 2 * NU))
P_RHO = 1.0
P_VOL = DX ** 3
P_MASS = P_RHO * P_VOL
EPSILON = 1e-14

_G3 = N_GRID ** 3
_ROWS = _G3 // N_GRID
_BLK = 2048


def _p2g_fields(state):
    x = state[:, 0:3]
    v = state[:, 3:6]
    C = state[:, 6:15].reshape(-1, 3, 3)
    F = state[:, 15:24].reshape(-1, 3, 3)
    x_c = jnp.clip(x, 0.5 * DX, 1.0 - 0.5 * DX)
    base_f = x_c * INV_DX - 0.5
    base = base_f.astype(jnp.int32)
    fx = x_c * INV_DX - base.astype(jnp.float32)
    w = jnp.stack([0.5 * (1.5 - fx) ** 2,
                   0.75 - (fx - 1.0) ** 2,
                   0.5 * (fx - 0.5) ** 2], axis=0)
    J = jnp.abs(jnp.linalg.det(F))
    s = -DT * P_VOL * 4.0 * INV_DX * INV_DX * LA * J * (J - 1.0)
    eye = jnp.eye(3, dtype=jnp.float32)
    affine = s[:, None, None] * eye[None] + P_MASS * C
    base = jnp.clip(base, 0, N_GRID - 3)
    grid_v = jnp.zeros((_G3, 3), dtype=jnp.float32)
    grid_m = jnp.zeros((_G3,), dtype=jnp.float32)
    for i in range(3):
        for j in range(3):
            for k in range(3):
                offset = jnp.array([i, j, k], dtype=jnp.float32)
                dpos = (offset[None, :] - fx) * DX
                weight = w[i, :, 0] * w[j, :, 1] * w[k, :, 2]
                lin = (base[:, 0] + i) * N_GRID * N_GRID + (base[:, 1] + j) * N_GRID + (base[:, 2] + k)
                contrib = weight[:, None] * (P_MASS * v + jnp.einsum('nij,nj->ni', affine, dpos))
                grid_v = grid_v.at[lin].add(contrib)
                grid_m = grid_m.at[lin].add(weight * P_MASS)
    return grid_v, grid_m


def _loss_body(mp_ref, mt_ref, dx_ref, dy_ref, dz_ref, om_ref, ov_ref, oc_ref):
    mp = mp_ref[...]
    mt = mt_ref[...]
    mask = (mp > EPSILON) | (mt > EPSILON)
    dm = mp - mt
    zero = jnp.zeros_like(mp)
    lm = jnp.sum(jnp.where(mask, dm * dm, zero))
    vv = dx_ref[...] ** 2 + dy_ref[...] ** 2 + dz_ref[...] ** 2
    lv = jnp.sum(jnp.where(mask, vv, zero))
    cnt = jnp.sum(mask.astype(jnp.float32))

    @pl.when(pl.program_id(0) == 0)
    def _init():
        om_ref[...] = lm.reshape(1, 1)
        ov_ref[...] = lv.reshape(1, 1)
        oc_ref[...] = cnt.reshape(1, 1)

    @pl.when(pl.program_id(0) != 0)
    def _acc():
        om_ref[...] += lm.reshape(1, 1)
        ov_ref[...] += lv.reshape(1, 1)
        oc_ref[...] += cnt.reshape(1, 1)


@functools.partial(jax.jit)
def _masked_loss(gm_pre, gm_tru, dvx, dvy, dvz):
    spec = pl.BlockSpec((_BLK, N_GRID), lambda g: (g, 0))
    out = pl.pallas_call(
        _loss_body,
        grid=(_ROWS // _BLK,),
        in_specs=[spec] * 5,
        out_specs=[pl.BlockSpec((1, 1), lambda g: (0, 0))] * 3,
        out_shape=[jax.ShapeDtypeStruct((1, 1), jnp.float32)] * 3,
    )(gm_pre, gm_tru, dvx, dvy, dvz)
    lm, lv, cnt = out[0][0, 0], out[1][0, 0], out[2][0, 0]
    loss_m = 1e11 * lm / jnp.maximum(cnt, 1.0)
    loss_v = 1e11 * lv / jnp.maximum(cnt, EPSILON)
    return loss_m + loss_v


def kernel(x_pred, x_true, R_pred, R_true):
    del R_pred, R_true
    gv_pre, gm_pre = _p2g_fields(x_pred)
    gv_tru, gm_tru = _p2g_fields(x_true)
    dv = gv_pre - gv_tru
    shp = (_ROWS, N_GRID)
    return _masked_loss(gm_pre.reshape(shp), gm_tru.reshape(shp),
                        dv[:, 0].reshape(shp), dv[:, 1].reshape(shp),
                        dv[:, 2].reshape(shp))


# speedup vs baseline: 60.4082x; 50.5161x over previous
import functools

import jax
import jax.numpy as jnp
from jax import lax
from jax.experimental import pallas as pl
from jax.experimental.pallas import tpu as pltpu
from jax.experimental.pallas import tpu_sc as plsc

N_PART = 113724
N_GRID = 128
DT = 5e-05
DX = 1.0 / N_GRID
INV_DX = float(N_GRID)
E_MOD = 1200.0
NU = 0.3
LA = E_MOD * NU / ((1 + NU) * (1 - 2 * NU))
P_RHO = 1.0
P_VOL = DX ** 3
P_MASS = P_RHO * P_VOL
EPSILON = 1e-14
K_PRESS = -DT * P_VOL * 4.0 * INV_DX * INV_DX * LA

G3 = N_GRID ** 3
HALF = G3 // 2
_ROWS = G3 // N_GRID
_RBLK = 2048

NC, NS, L = 2, 16, 16
W = 256
GR = W // L
NWIN = 28
PT = NWIN * W
N_PAD = NS * PT
PAIRS = 27 * W
ZCH = 8192
QN = HALF // NS // ZCH

_PBLK = 2048
_OFFS = [(i, j, k) for i in range(3) for j in range(3) for k in range(3)]
_OFFL = [i * N_GRID * N_GRID + j * N_GRID + k for (i, j, k) in _OFFS]


def _builder_body(xt_ref, rec_ref, lin_ref):
    col0 = pl.program_id(0) * _PBLK
    gidx = col0 + lax.broadcasted_iota(jnp.int32, (1, _PBLK), 1)
    live = gidx < N_PART

    def row(r):
        return xt_ref[r:r + 1, :]

    fx = []
    base = []
    for d in range(3):
        xc = jnp.clip(row(d), 0.5 * DX, 1.0 - 0.5 * DX)
        t = xc * INV_DX
        bi = (t - 0.5).astype(jnp.int32)
        fxd = t - bi.astype(jnp.float32)
        fx.append(fxd)
        base.append(bi)
        w0 = 0.5 * (1.5 - fxd) ** 2
        w1 = 0.75 - (fxd - 1.0) ** 2
        w2 = 0.5 * (fxd - 0.5) ** 2
        if d == 0:
            livef = live.astype(jnp.float32)
            w0, w1, w2 = w0 * livef, w1 * livef, w2 * livef
        rec_ref[3 * d:3 * d + 1, :] = w0
        rec_ref[3 * d + 1:3 * d + 2, :] = w1
        rec_ref[3 * d + 2:3 * d + 3, :] = w2

    f = [[row(15 + 3 * a + b) for b in range(3)] for a in range(3)]
    det = (f[0][0] * (f[1][1] * f[2][2] - f[1][2] * f[2][1])
           - f[0][1] * (f[1][0] * f[2][2] - f[1][2] * f[2][0])
           + f[0][2] * (f[1][0] * f[2][1] - f[1][1] * f[2][0]))
    jdet = jnp.abs(det)
    s = K_PRESS * jdet * (jdet - 1.0)

    for c in range(3):
        brow = []
        for d in range(3):
            bcd = DX * ((s if c == d else 0.0) + P_MASS * row(6 + 3 * c + d))
            rec_ref[12 + 3 * c + d:13 + 3 * c + d, :] = bcd
            brow.append(bcd)
        a_c = (P_MASS * row(3 + c)
               - (brow[0] * fx[0] + brow[1] * fx[1] + brow[2] * fx[2]))
        rec_ref[9 + c:10 + c, :] = a_c

    bcl = [jnp.minimum(b, N_GRID - 3) for b in base]
    lin = bcl[0] * (N_GRID * N_GRID) + bcl[1] * N_GRID + bcl[2]
    spread = (gidx * 2053) & (G3 - 1)
    lin_ref[...] = jnp.where(live, lin, spread)


@jax.jit
def _build_records(xt):
    return pl.pallas_call(
        _builder_body,
        grid=(N_PAD // _PBLK,),
        in_specs=[pl.BlockSpec((24, _PBLK), lambda g: (0, g))],
        out_specs=[pl.BlockSpec((24, _PBLK), lambda g: (0, g)),
                   pl.BlockSpec((1, _PBLK), lambda g: (0, g))],
        out_shape=[jax.ShapeDtypeStruct((24, N_PAD), jnp.float32),
                   jax.ShapeDtypeStruct((1, N_PAD), jnp.int32)],
    )(xt)


def _sc_body(rec_p, lin_p, rec_t, lin_t,
             om_p, om_t, ox, oy, oz,
             grid_sh, recb, linb, pib, pvb, zb):
    cid = lax.axis_index("c")
    sid = lax.axis_index("s")
    base_cell = cid * HALF
    zero16 = jnp.zeros((L,), jnp.float32)

    def zinit(i, c):
        zb[pl.ds(i * L, L)] = zero16
        return c
    lax.fori_loop(0, ZCH // L, zinit, 0)

    def make_group_body(comp, sign):
        def group_body(g, c):
            sl = pl.ds(g * L, L)
            bl = linb[sl] - base_cell
            wx = [recb[r, sl] for r in range(0, 3)]
            wy = [recb[r, sl] for r in range(3, 6)]
            wz = [recb[r, sl] for r in range(6, 9)]
            wxy = [wx[i] * wy[j] for i in range(3) for j in range(3)]
            if comp == 'm':
                tv = None
            else:
                a = recb[9 + comp, sl]
                b0 = recb[12 + 3 * comp, sl]
                b1 = recb[13 + 3 * comp, sl]
                b2 = recb[14 + 3 * comp, sl]
                if sign < 0:
                    a, b0, b1, b2 = -a, -b0, -b1, -b2
                u = [a, a + b0, a + b0 + b0]
                tij = []
                for i in range(3):
                    tij.append(u[i])
                    tij.append(u[i] + b1)
                    tij.append(tij[-1] + b1)
                tv = []
                for ij in range(9):
                    tv.append(tij[ij])
                    tv.append(tv[-1] + b2)
                    tv.append(tv[-1] + b2)
            for o, (i, j, k) in enumerate(_OFFS):
                w = wxy[3 * i + j] * wz[k]
                val = w * P_MASS if comp == 'm' else w * tv[9 * i + 3 * j + k]
                loc = bl + _OFFL[o]
                inb = (loc >= 0) & (loc < HALF)
                pib[pl.ds(o * W + g * L, L)] = loc & (HALF - 1)
                pvb[pl.ds(o * W + g * L, L)] = jnp.where(inb, val, 0.0)
            return c
        return group_body

    def make_window_body(rec_hbm, lin_hbm, comp, sign):
        gbody = make_group_body(comp, sign)

        def window_body(wi, c):
            start = sid * PT + wi * W
            pltpu.sync_copy(rec_hbm.at[:, pl.ds(start, W)], recb)
            pltpu.sync_copy(lin_hbm.at[pl.ds(start, W)], linb)
            lax.fori_loop(0, GR, gbody, 0)
            pltpu.sync_copy(pvb, grid_sh.at[pib], add=True)
            return c
        return window_body

    schedule = [
        (om_p, [(rec_p, lin_p, 'm', 1)]),
        (om_t, [(rec_t, lin_t, 'm', 1)]),
        (ox, [(rec_p, lin_p, 0, 1), (rec_t, lin_t, 0, -1)]),
        (oy, [(rec_p, lin_p, 1, 1), (rec_t, lin_t, 1, -1)]),
        (oz, [(rec_p, lin_p, 2, 1), (rec_t, lin_t, 2, -1)]),
    ]
    for out_ref, passes in schedule:
        for q in range(QN):
            pltpu.sync_copy(zb, grid_sh.at[pl.ds(sid * (QN * ZCH) + q * ZCH, ZCH)])
        plsc.subcore_barrier()
        for (rec_hbm, lin_hbm, comp, sign) in passes:
            lax.fori_loop(0, NWIN, make_window_body(rec_hbm, lin_hbm, comp, sign), 0)
        plsc.subcore_barrier()
        for q in range(QN):
            off = sid * (QN * ZCH) + q * ZCH
            pltpu.sync_copy(grid_sh.at[pl.ds(off, ZCH)],
                            out_ref.at[pl.ds(base_cell + off, ZCH)])
        plsc.subcore_barrier()


_sc_scatter = functools.partial(
    pl.kernel,
    out_type=[jax.ShapeDtypeStruct((G3,), jnp.float32)] * 5,
    mesh=plsc.VectorSubcoreMesh(core_axis_name="c", subcore_axis_name="s"),
    scratch_types=[
        pltpu.VMEM_SHARED((HALF,), jnp.float32),
        pltpu.VMEM((24, W), jnp.float32),
        pltpu.VMEM((W,), jnp.int32),
        pltpu.VMEM((PAIRS,), jnp.int32),
        pltpu.VMEM((PAIRS,), jnp.float32),
        pltpu.VMEM((ZCH,), jnp.float32),
    ],
)(_sc_body)


def _loss_body(mp_ref, mt_ref, dx_ref, dy_ref, dz_ref, om_ref, ov_ref, oc_ref):
    mp = mp_ref[...]
    mt = mt_ref[...]
    mask = (mp > EPSILON) | (mt > EPSILON)
    dm = mp - mt
    zero = jnp.zeros_like(mp)
    lm = jnp.sum(jnp.where(mask, dm * dm, zero))
    vv = dx_ref[...] ** 2 + dy_ref[...] ** 2 + dz_ref[...] ** 2
    lv = jnp.sum(jnp.where(mask, vv, zero))
    cnt = jnp.sum(mask.astype(jnp.float32))

    @pl.when(pl.program_id(0) == 0)
    def _init():
        om_ref[...] = lm.reshape(1, 1)
        ov_ref[...] = lv.reshape(1, 1)
        oc_ref[...] = cnt.reshape(1, 1)

    @pl.when(pl.program_id(0) != 0)
    def _acc():
        om_ref[...] += lm.reshape(1, 1)
        ov_ref[...] += lv.reshape(1, 1)
        oc_ref[...] += cnt.reshape(1, 1)


def _masked_loss(gm_pre, gm_tru, dvx, dvy, dvz):
    spec = pl.BlockSpec((_RBLK, N_GRID), lambda g: (g, 0))
    out = pl.pallas_call(
        _loss_body,
        grid=(_ROWS // _RBLK,),
        in_specs=[spec] * 5,
        out_specs=[pl.BlockSpec((1, 1), lambda g: (0, 0))] * 3,
        out_shape=[jax.ShapeDtypeStruct((1, 1), jnp.float32)] * 3,
    )(gm_pre, gm_tru, dvx, dvy, dvz)
    lm, lv, cnt = out[0][0, 0], out[1][0, 0], out[2][0, 0]
    loss_m = 1e11 * lm / jnp.maximum(cnt, 1.0)
    loss_v = 1e11 * lv / jnp.maximum(cnt, EPSILON)
    return loss_m + loss_v


@jax.jit
def _run(x_pred, x_true):
    recs = []
    for x in (x_pred, x_true):
        xp = jnp.pad(x, ((0, N_PAD - N_PART), (0, 0)))
        rec, lin = _build_records(xp.T)
        recs.append((rec, lin.reshape(N_PAD)))
    gm_p, gm_t, dvx, dvy, dvz = _sc_scatter(
        recs[0][0], recs[0][1], recs[1][0], recs[1][1])
    shp = (_ROWS, N_GRID)
    return _masked_loss(gm_p.reshape(shp), gm_t.reshape(shp),
                        dvx.reshape(shp), dvy.reshape(shp), dvz.reshape(shp))


def kernel(x_pred, x_true, R_pred, R_true):
    del R_pred, R_true
    return _run(x_pred, x_true)
